# baseline (speedup 1.0000x reference)
import functools

import jax
import jax.numpy as jnp
from jax import lax
from jax.experimental import pallas as pl
from jax.experimental.pallas import tpu as pltpu
from jax.experimental.pallas import tpu_sc as plsc

_ROWS = 64
_DIM = 128
_BATCH = 16384
_NC, _NS = 2, 16
_NW = _NC * _NS
_BPW = _BATCH // _NW
_CHUNK = 128
_NCHUNK = _BPW // _CHUNK
_RPT = _ROWS // _NS
_L = 16

_mesh = plsc.VectorSubcoreMesh(
    core_axis_name="c", subcore_axis_name="s", num_cores=_NC, num_subcores=_NS
)


def _rsqrt16(x):
    i = plsc.bitcast(x, jnp.int32)
    i = jnp.full((_L,), 0x5F3759DF, jnp.int32) - (i >> 1)
    y = plsc.bitcast(i, jnp.float32)
    for _ in range(3):
        y = y * (1.5 - 0.5 * x * y * y)
    return y


@functools.partial(
    pl.kernel,
    out_type=jax.ShapeDtypeStruct((_BATCH, _DIM), jnp.float32),
    mesh=_mesh,
    scratch_types=[
        pltpu.VMEM((_NCHUNK, _CHUNK), jnp.int32),
        pltpu.VMEM((_NCHUNK, _CHUNK, _DIM), jnp.float32),
        pltpu.VMEM((_RPT, _DIM), jnp.float32),
        pltpu.VMEM((_RPT, _DIM), jnp.float32),
        pltpu.VMEM((_RPT * _L,), jnp.float32),
        pltpu.VMEM((_RPT * _L,), jnp.int32),
        pltpu.VMEM((_L,), jnp.float32),
        pltpu.VMEM((_RPT, _L), jnp.float32),
        pltpu.VMEM_SHARED((_ROWS, _DIM), jnp.float32),
        pltpu.VMEM_SHARED((_NS * _L,), jnp.float32),
        pltpu.SemaphoreType.DMA,
        pltpu.SemaphoreType.DMA,
        pltpu.SemaphoreType.DMA,
        [pltpu.SemaphoreType.DMA] * _NCHUNK,
        pltpu.SemaphoreType.DMA,
    ],
)
def _codebook(tab_hbm, idx_hbm, out_hbm, idx_v, rows_v, raw_v, nrm_v, acc_v,
              ridx_v, vbuf_v, splat_v, stab, sred, isem, rsem, psem, gsems,
              ssem):
    sid = lax.axis_index("s")
    wid = sid * _NC + lax.axis_index("c")
    base = wid * _BPW
    s16 = sid * _L

    icp = pltpu.async_copy(idx_hbm.at[wid], idx_v, isem)
    rcp = pltpu.async_copy(tab_hbm.at[pl.ds(sid * _RPT, _RPT)], raw_v, rsem)

    vbuf_v[...] = jnp.zeros((_L,), jnp.float32)
    pltpu.sync_copy(vbuf_v, sred.at[pl.ds(s16, _L)])

    rcp.wait()
    for r in range(_RPT):
        acc = jnp.zeros((_L,), jnp.float32)
        for j in range(_DIM // _L):
            chunk = raw_v[r, pl.ds(j * _L, _L)]
            acc = acc + chunk * chunk
        acc_v[pl.ds(r * _L, _L)] = acc
        ridx_v[pl.ds(r * _L, _L)] = jnp.full((_L,), s16 + r, jnp.int32)
    pltpu.sync_copy(acc_v, sred.at[ridx_v], add=True)
    pcps = [
        pltpu.async_copy(
            sred.at[ridx_v.at[pl.ds(r * _L, _L)]], splat_v.at[r], psem)
        for r in range(_RPT)
    ]
    for cp in pcps:
        cp.wait()
    for r in range(_RPT):
        x = jnp.maximum(splat_v[r, ...], 1e-24)
        s = 0.5 * (1.0 + x)
        for _ in range(8):
            s = 0.5 * (s + x / s)
        inv = 1.0 / s
        for j in range(_DIM // _L):
            nrm_v[r, pl.ds(j * _L, _L)] = raw_v[r, pl.ds(j * _L, _L)] * inv
    pltpu.sync_copy(nrm_v, stab.at[pl.ds(sid * _RPT, _RPT)])
    plsc.subcore_barrier()

    icp.wait()
    gcps = [
        pltpu.async_copy(stab.at[idx_v.at[g]], rows_v.at[g], gsems[g])
        for g in range(_NCHUNK)
    ]
    scps = []
    for g in range(_NCHUNK):
        gcps[g].wait()
        scps.append(
            pltpu.async_copy(
                rows_v.at[g], out_hbm.at[pl.ds(base + g * _CHUNK, _CHUNK)], ssem
            )
        )
    for c in scps:
        c.wait()


def kernel(indices, table):
    idx3 = indices.astype(jnp.int32).reshape(_NW, _NCHUNK, _CHUNK)
    return _codebook(table, idx3)


# speedup vs baseline: 1.0164x; 1.0164x over previous
import functools

import jax
import jax.numpy as jnp
from jax import lax
from jax.experimental import pallas as pl
from jax.experimental.pallas import tpu as pltpu
from jax.experimental.pallas import tpu_sc as plsc

_ROWS = 64
_DIM = 128
_BATCH = 16384
_NC, _NS = 2, 16
_NW = _NC * _NS
_BPW = _BATCH // _NW
_CHUNK = 128
_NCHUNK = _BPW // _CHUNK
_RPT = _ROWS // _NS
_L = 16

_mesh = plsc.VectorSubcoreMesh(
    core_axis_name="c", subcore_axis_name="s", num_cores=_NC, num_subcores=_NS
)


def _rsqrt16(x):
    i = plsc.bitcast(x, jnp.int32)
    i = jnp.full((_L,), 0x5F3759DF, jnp.int32) - (i >> 1)
    y = plsc.bitcast(i, jnp.float32)
    for _ in range(3):
        y = y * (1.5 - 0.5 * x * y * y)
    return y


@functools.partial(
    pl.kernel,
    out_type=jax.ShapeDtypeStruct((_BATCH, _DIM), jnp.float32),
    mesh=_mesh,
    scratch_types=[
        pltpu.VMEM((_NCHUNK, _CHUNK), jnp.int32),
        pltpu.VMEM((_NCHUNK, _CHUNK, _DIM), jnp.float32),
        pltpu.VMEM((_RPT, _DIM), jnp.float32),
        pltpu.VMEM((_RPT, _DIM), jnp.float32),
        pltpu.VMEM((_RPT * _L,), jnp.float32),
        pltpu.VMEM((_RPT * _L,), jnp.int32),
        pltpu.VMEM((_L,), jnp.float32),
        pltpu.VMEM((_RPT, _L), jnp.float32),
        pltpu.VMEM_SHARED((_ROWS, _DIM), jnp.float32),
        pltpu.VMEM_SHARED((_NS * _L,), jnp.float32),
        pltpu.SemaphoreType.DMA,
        pltpu.SemaphoreType.DMA,
        pltpu.SemaphoreType.DMA,
        [pltpu.SemaphoreType.DMA] * _NCHUNK,
        pltpu.SemaphoreType.DMA,
    ],
)
def _codebook(tab_hbm, idx_hbm, out_hbm, idx_v, rows_v, raw_v, nrm_v, acc_v,
              ridx_v, vbuf_v, splat_v, stab, sred, isem, rsem, psem, gsems,
              ssem):
    sid = lax.axis_index("s")
    wid = sid * _NC + lax.axis_index("c")
    base = wid * _BPW
    s16 = sid * _L

    icp = pltpu.async_copy(idx_hbm.at[wid], idx_v, isem)
    rcp = pltpu.async_copy(tab_hbm.at[pl.ds(sid * _RPT, _RPT)], raw_v, rsem)

    vbuf_v[...] = jnp.zeros((_L,), jnp.float32)
    pltpu.sync_copy(vbuf_v, sred.at[pl.ds(s16, _L)])

    rcp.wait()
    for r in range(_RPT):
        acc = jnp.zeros((_L,), jnp.float32)
        for j in range(_DIM // _L):
            chunk = raw_v[r, pl.ds(j * _L, _L)]
            acc = acc + chunk * chunk
        acc_v[pl.ds(r * _L, _L)] = acc
        ridx_v[pl.ds(r * _L, _L)] = jnp.full((_L,), s16 + r, jnp.int32)
    pltpu.sync_copy(acc_v, sred.at[ridx_v], add=True)
    pcps = [
        pltpu.async_copy(
            sred.at[ridx_v.at[pl.ds(r * _L, _L)]], splat_v.at[r], psem)
        for r in range(_RPT)
    ]
    for cp in pcps:
        cp.wait()
    for r in range(_RPT):
        x = jnp.maximum(splat_v[r, ...], 1e-24)
        y = 1.0 / (0.4 + 0.4 * x)
        for _ in range(4):
            y = y * (1.5 - 0.5 * x * y * y)
        inv = y
        for j in range(_DIM // _L):
            nrm_v[r, pl.ds(j * _L, _L)] = raw_v[r, pl.ds(j * _L, _L)] * inv
    pltpu.sync_copy(nrm_v, stab.at[pl.ds(sid * _RPT, _RPT)])
    plsc.subcore_barrier()

    icp.wait()
    gcps = [
        pltpu.async_copy(stab.at[idx_v.at[g]], rows_v.at[g], gsems[g])
        for g in range(_NCHUNK)
    ]
    scps = []
    for g in range(_NCHUNK):
        gcps[g].wait()
        scps.append(
            pltpu.async_copy(
                rows_v.at[g], out_hbm.at[pl.ds(base + g * _CHUNK, _CHUNK)], ssem
            )
        )
    for c in scps:
        c.wait()


def kernel(indices, table):
    idx3 = indices.astype(jnp.int32).reshape(_NW, _NCHUNK, _CHUNK)
    return _codebook(table, idx3)


# speedup vs baseline: 1.0422x; 1.0254x over previous
import functools

import jax
import jax.numpy as jnp
from jax import lax
from jax.experimental import pallas as pl
from jax.experimental.pallas import tpu as pltpu
from jax.experimental.pallas import tpu_sc as plsc

_ROWS = 64
_DIM = 128
_BATCH = 16384
_NC, _NS = 2, 16
_NW = _NC * _NS
_BPW = _BATCH // _NW
_CHUNK = 128
_NCHUNK = _BPW // _CHUNK


def _normalize_body(tab_ref, out_ref):
    t = tab_ref[...]
    ssq = jnp.sum(t * t, axis=1, keepdims=True)
    out_ref[...] = t * lax.rsqrt(jnp.maximum(ssq, 1e-24))


_normalize = pl.pallas_call(
    _normalize_body,
    out_shape=jax.ShapeDtypeStruct((_ROWS, _DIM), jnp.float32),
)

_mesh = plsc.VectorSubcoreMesh(
    core_axis_name="c", subcore_axis_name="s", num_cores=_NC, num_subcores=_NS
)


@functools.partial(
    pl.kernel,
    out_type=jax.ShapeDtypeStruct((_BATCH, _DIM), jnp.float32),
    mesh=_mesh,
    scratch_types=[
        pltpu.VMEM((_NCHUNK, _CHUNK), jnp.int32),
        pltpu.VMEM((_NCHUNK, _CHUNK, _DIM), jnp.float32),
        pltpu.VMEM_SHARED((_ROWS, _DIM), jnp.float32),
        pltpu.SemaphoreType.DMA,
        pltpu.SemaphoreType.DMA,
        [pltpu.SemaphoreType.DMA] * _NCHUNK,
        pltpu.SemaphoreType.DMA,
    ],
)
def _gather(ntab_hbm, idx_hbm, out_hbm, idx_v, rows_v, stab, isem, tsem,
            gsems, ssem):
    sid = lax.axis_index("s")
    wid = sid * _NC + lax.axis_index("c")
    base = wid * _BPW

    _RPT = _ROWS // _NS
    icp = pltpu.async_copy(idx_hbm.at[wid], idx_v, isem)
    tcp = pltpu.async_copy(
        ntab_hbm.at[pl.ds(sid * _RPT, _RPT)],
        stab.at[pl.ds(sid * _RPT, _RPT)],
        tsem,
    )
    tcp.wait()
    plsc.subcore_barrier()
    icp.wait()

    gcps = [
        pltpu.async_copy(stab.at[idx_v.at[g]], rows_v.at[g], gsems[g])
        for g in range(_NCHUNK)
    ]
    scps = []
    for g in range(_NCHUNK):
        gcps[g].wait()
        scps.append(
            pltpu.async_copy(
                rows_v.at[g], out_hbm.at[pl.ds(base + g * _CHUNK, _CHUNK)], ssem
            )
        )
    for c in scps:
        c.wait()


def kernel(indices, table):
    ntab = _normalize(table)
    idx3 = indices.astype(jnp.int32).reshape(_NW, _NCHUNK, _CHUNK)
    return _gather(ntab, idx3)
